# baseline (device time: 121435 ns/iter reference)
import jax
import jax.numpy as jnp
from jax import lax
from jax.experimental import pallas as pl
from jax.experimental.pallas import tpu as pltpu

M, N = 4096, 1024
NRING = 16
C = 136
DIRECT = M - NRING * C

CYCLE = [
    (0, 0), (1, 0), (2, 0), (3, 0),
    (3, 1), (2, 1), (1, 1), (1, 2),
    (2, 2), (3, 2), (3, 3), (2, 3),
    (1, 3), (0, 3), (0, 2), (0, 1),
]
POS_BY_YZ = [0] * 16
for _p, (_y, _z) in enumerate(CYCLE):
    POS_BY_YZ[_y * 4 + _z] = _p
RIGHT_Y = [CYCLE[(_p + 1) % NRING][0] for _p in range(NRING)]
RIGHT_Z = [CYCLE[(_p + 1) % NRING][1] for _p in range(NRING)]
LEFT_Y = [CYCLE[(_p - 1) % NRING][0] for _p in range(NRING)]
LEFT_Z = [CYCLE[(_p - 1) % NRING][1] for _p in range(NRING)]

CW_HOPS = 8
CCW_HOPS = 7

_MESH = pl.DeviceIdType.MESH


def _lut(idx, table):
    acc = jnp.int32(table[0])
    for k in range(1, len(table)):
        acc = jnp.where(idx == k, jnp.int32(table[k]), acc)
    return acc


def kernel(x):
    def body(
        x_ref,
        out_ref,
        direct_recv,
        chunk_recv,
        ring_buf,
        direct_send_sem,
        direct_recv_sem,
        chunk_send_sem,
        chunk_recv_sem,
        cw_send_sems,
        cw_recv_sems,
        ccw_send_sems,
        ccw_recv_sems,
    ):
        my_x = lax.axis_index("x")
        my_y = lax.axis_index("y")
        my_z = lax.axis_index("z")
        partner = (1 - my_x, my_y, my_z)
        p = _lut(my_y * 4 + my_z, POS_BY_YZ)
        right = (my_x, _lut(p, RIGHT_Y), _lut(p, RIGHT_Z))
        left = (my_x, _lut(p, LEFT_Y), _lut(p, LEFT_Z))

        barrier = pltpu.get_barrier_semaphore()
        for nbr in (partner, right, left):
            pl.semaphore_signal(barrier, inc=1, device_id=nbr,
                                device_id_type=_MESH)
        pl.semaphore_wait(barrier, 3)

        chunk_rdma = pltpu.make_async_remote_copy(
            src_ref=x_ref.at[pl.ds(DIRECT + p * C, C), :],
            dst_ref=chunk_recv,
            send_sem=chunk_send_sem,
            recv_sem=chunk_recv_sem,
            device_id=partner,
            device_id_type=_MESH,
        )
        chunk_rdma.start()
        chunk_rdma.wait()

        dhalf = DIRECT // 2
        direct_rdmas = []
        for i in range(2):
            d = pltpu.make_async_remote_copy(
                src_ref=x_ref.at[pl.ds(i * dhalf, dhalf), :],
                dst_ref=direct_recv.at[pl.ds(i * dhalf, dhalf), :],
                send_sem=direct_send_sem.at[i],
                recv_sem=direct_recv_sem.at[i],
                device_id=partner,
                device_id_type=_MESH,
            )
            d.start()
            direct_rdmas.append(d)

        ring_buf[p] = x_ref[pl.ds(DIRECT + p * C, C), :] + chunk_recv[...]

        def _cw_send(h):
            slot = jnp.remainder(p - h, NRING)
            d = pltpu.make_async_remote_copy(
                src_ref=ring_buf.at[slot],
                dst_ref=ring_buf.at[slot],
                send_sem=cw_send_sems.at[h],
                recv_sem=cw_recv_sems.at[h],
                device_id=right,
                device_id_type=_MESH,
            )
            d.start()
            return d

        def _ccw_send(h):
            slot = jnp.remainder(p + h, NRING)
            d = pltpu.make_async_remote_copy(
                src_ref=ring_buf.at[slot],
                dst_ref=ring_buf.at[slot],
                send_sem=ccw_send_sems.at[h],
                recv_sem=ccw_recv_sems.at[h],
                device_id=left,
                device_id_type=_MESH,
            )
            d.start()
            return d

        def _wait_recv(origin, recv_sems, h):
            pltpu.make_async_remote_copy(
                src_ref=ring_buf.at[origin],
                dst_ref=ring_buf.at[origin],
                send_sem=cw_send_sems.at[0],
                recv_sem=recv_sems.at[h],
                device_id=left,
                device_id_type=_MESH,
            ).wait_recv()

        send_descs = [_cw_send(0), _ccw_send(0)]
        out_ref[pl.ds(DIRECT + p * C, C), :] = ring_buf[p]
        for h in range(CW_HOPS):
            o_cw = jnp.remainder(p - h - 1, NRING)
            _wait_recv(o_cw, cw_recv_sems, h)
            if h + 1 < CW_HOPS:
                send_descs.append(_cw_send(h + 1))
            o_ccw = None
            if h < CCW_HOPS:
                o_ccw = jnp.remainder(p + h + 1, NRING)
                _wait_recv(o_ccw, ccw_recv_sems, h)
                if h + 1 < CCW_HOPS:
                    send_descs.append(_ccw_send(h + 1))
            out_ref[pl.ds(DIRECT + o_cw * C, C), :] = ring_buf[o_cw]
            if o_ccw is not None:
                out_ref[pl.ds(DIRECT + o_ccw * C, C), :] = ring_buf[o_ccw]

        for i, d in enumerate(direct_rdmas):
            d.wait_recv()
            out_ref[pl.ds(i * dhalf, dhalf), :] = (
                x_ref[pl.ds(i * dhalf, dhalf), :]
                + direct_recv[pl.ds(i * dhalf, dhalf), :]
            )
        for d in direct_rdmas:
            d.wait_send()
        for d in send_descs:
            d.wait_send()

    return pl.pallas_call(
        body,
        out_shape=jax.ShapeDtypeStruct((M, N), x.dtype),
        in_specs=[pl.BlockSpec(memory_space=pltpu.VMEM)],
        out_specs=pl.BlockSpec(memory_space=pltpu.VMEM),
        scratch_shapes=[
            pltpu.VMEM((DIRECT, N), x.dtype),
            pltpu.VMEM((C, N), x.dtype),
            pltpu.VMEM((NRING, C, N), x.dtype),
            pltpu.SemaphoreType.DMA((2,)),
            pltpu.SemaphoreType.DMA((2,)),
            pltpu.SemaphoreType.DMA,
            pltpu.SemaphoreType.DMA,
            pltpu.SemaphoreType.DMA((CW_HOPS,)),
            pltpu.SemaphoreType.DMA((CW_HOPS,)),
            pltpu.SemaphoreType.DMA((CCW_HOPS,)),
            pltpu.SemaphoreType.DMA((CCW_HOPS,)),
        ],
        compiler_params=pltpu.CompilerParams(
            collective_id=0, vmem_limit_bytes=100 * 1024 * 1024
        ),
    )(x)


# device time: 105096 ns/iter; 1.1555x vs baseline; 1.1555x over previous
import jax
import jax.numpy as jnp
from jax import lax
from jax.experimental import pallas as pl
from jax.experimental.pallas import tpu as pltpu

M, N = 4096, 1024
NRING = 16
C = 160
SUB = 2
CS = C // SUB
DIRECT = M - NRING * C

CYCLE = [
    (0, 0), (1, 0), (2, 0), (3, 0),
    (3, 1), (2, 1), (1, 1), (1, 2),
    (2, 2), (3, 2), (3, 3), (2, 3),
    (1, 3), (0, 3), (0, 2), (0, 1),
]
POS_BY_YZ = [0] * 16
for _p, (_y, _z) in enumerate(CYCLE):
    POS_BY_YZ[_y * 4 + _z] = _p
RIGHT_Y = [CYCLE[(_p + 1) % NRING][0] for _p in range(NRING)]
RIGHT_Z = [CYCLE[(_p + 1) % NRING][1] for _p in range(NRING)]
LEFT_Y = [CYCLE[(_p - 1) % NRING][0] for _p in range(NRING)]
LEFT_Z = [CYCLE[(_p - 1) % NRING][1] for _p in range(NRING)]

CW_HOPS = 8 * SUB
CCW_HOPS = 7 * SUB

_MESH = pl.DeviceIdType.MESH


def _lut(idx, table):
    acc = jnp.int32(table[0])
    for k in range(1, len(table)):
        acc = jnp.where(idx == k, jnp.int32(table[k]), acc)
    return acc


def kernel(x):
    def body(
        x_ref,
        out_ref,
        direct_recv,
        chunk_recv,
        ring_buf,
        direct_send_sem,
        direct_recv_sem,
        chunk_send_sems,
        chunk_recv_sems,
        cw_send_sems,
        cw_recv_sems,
        ccw_send_sems,
        ccw_recv_sems,
    ):
        my_x = lax.axis_index("x")
        my_y = lax.axis_index("y")
        my_z = lax.axis_index("z")
        partner = (1 - my_x, my_y, my_z)
        p = _lut(my_y * 4 + my_z, POS_BY_YZ)
        right = (my_x, _lut(p, RIGHT_Y), _lut(p, RIGHT_Z))
        left = (my_x, _lut(p, LEFT_Y), _lut(p, LEFT_Z))

        barrier = pltpu.get_barrier_semaphore()
        for nbr in (partner, right, left):
            pl.semaphore_signal(barrier, inc=1, device_id=nbr,
                                device_id_type=_MESH)
        pl.semaphore_wait(barrier, 3)

        chunk_rdmas = []
        for s in range(SUB):
            d = pltpu.make_async_remote_copy(
                src_ref=x_ref.at[pl.ds(DIRECT + p * C + s * CS, CS), :],
                dst_ref=chunk_recv.at[pl.ds(s * CS, CS), :],
                send_sem=chunk_send_sems.at[s],
                recv_sem=chunk_recv_sems.at[s],
                device_id=partner,
                device_id_type=_MESH,
            )
            d.start()
            chunk_rdmas.append(d)

        direct_rdma = pltpu.make_async_remote_copy(
            src_ref=x_ref.at[pl.ds(0, DIRECT), :],
            dst_ref=direct_recv,
            send_sem=direct_send_sem,
            recv_sem=direct_recv_sem,
            device_id=partner,
            device_id_type=_MESH,
        )
        direct_rdma.start()

        def _sub(ref, o, s):
            return ref.at[o, pl.ds(s * CS, CS), :]

        def _send(o, s, sems_s, sems_r, t, dev):
            d = pltpu.make_async_remote_copy(
                src_ref=_sub(ring_buf, o, s),
                dst_ref=_sub(ring_buf, o, s),
                send_sem=sems_s.at[t],
                recv_sem=sems_r.at[t],
                device_id=dev,
                device_id_type=_MESH,
            )
            d.start()
            return d

        def _wait_recv(o, s, sems_r, t):
            pltpu.make_async_remote_copy(
                src_ref=_sub(ring_buf, o, s),
                dst_ref=_sub(ring_buf, o, s),
                send_sem=cw_send_sems.at[0],
                recv_sem=sems_r.at[t],
                device_id=left,
                device_id_type=_MESH,
            ).wait_recv()

        send_descs = []
        for s in range(SUB):
            chunk_rdmas[s].wait()
            rows = pl.ds(DIRECT + p * C + s * CS, CS)
            _sub(ring_buf, p, s)[...] = (
                x_ref[rows, :] + chunk_recv[pl.ds(s * CS, CS), :]
            )
            send_descs.append(_send(p, s, cw_send_sems, cw_recv_sems, s, right))
            send_descs.append(_send(p, s, ccw_send_sems, ccw_recv_sems, s, left))
        for s in range(SUB):
            out_ref[pl.ds(DIRECT + p * C + s * CS, CS), :] = _sub(
                ring_buf, p, s
            )[...]

        for t in range(CW_HOPS):
            s = t % SUB
            o_cw = jnp.remainder(p - 1 - t // SUB, NRING)
            _wait_recv(o_cw, s, cw_recv_sems, t)
            if t + SUB < CW_HOPS:
                send_descs.append(
                    _send(o_cw, s, cw_send_sems, cw_recv_sems, t + SUB, right)
                )
            o_ccw = None
            if t < CCW_HOPS:
                o_ccw = jnp.remainder(p + 1 + t // SUB, NRING)
                _wait_recv(o_ccw, s, ccw_recv_sems, t)
                if t + SUB < CCW_HOPS:
                    send_descs.append(
                        _send(o_ccw, s, ccw_send_sems, ccw_recv_sems,
                              t + SUB, left)
                    )
            out_ref[pl.ds(DIRECT + o_cw * C + s * CS, CS), :] = _sub(
                ring_buf, o_cw, s
            )[...]
            if o_ccw is not None:
                out_ref[pl.ds(DIRECT + o_ccw * C + s * CS, CS), :] = _sub(
                    ring_buf, o_ccw, s
                )[...]

        direct_rdma.wait_recv()
        out_ref[pl.ds(0, DIRECT), :] = (
            x_ref[pl.ds(0, DIRECT), :] + direct_recv[...]
        )
        direct_rdma.wait_send()
        for d in send_descs:
            d.wait_send()

    return pl.pallas_call(
        body,
        out_shape=jax.ShapeDtypeStruct((M, N), x.dtype),
        in_specs=[pl.BlockSpec(memory_space=pltpu.VMEM)],
        out_specs=pl.BlockSpec(memory_space=pltpu.VMEM),
        scratch_shapes=[
            pltpu.VMEM((DIRECT, N), x.dtype),
            pltpu.VMEM((C, N), x.dtype),
            pltpu.VMEM((NRING, C, N), x.dtype),
            pltpu.SemaphoreType.DMA,
            pltpu.SemaphoreType.DMA,
            pltpu.SemaphoreType.DMA((SUB,)),
            pltpu.SemaphoreType.DMA((SUB,)),
            pltpu.SemaphoreType.DMA((CW_HOPS,)),
            pltpu.SemaphoreType.DMA((CW_HOPS,)),
            pltpu.SemaphoreType.DMA((CCW_HOPS,)),
            pltpu.SemaphoreType.DMA((CCW_HOPS,)),
        ],
        compiler_params=pltpu.CompilerParams(
            collective_id=0, vmem_limit_bytes=100 * 1024 * 1024
        ),
    )(x)
